# Initial kernel scaffold; baseline (speedup 1.0000x reference)
import jax
import jax.numpy as jnp
from jax.experimental import pallas as pl


def kernel(x, W_in, W_rec, a, B_w, A_w, A_b, h0):
    raise NotImplementedError("write your pallas kernel here")


# speedup vs baseline: 14.9169x; 14.9169x over previous
import functools

import jax
import jax.numpy as jnp
from jax import lax
from jax.experimental import pallas as pl
from jax.experimental.pallas import tpu as pltpu
from jax.experimental.pallas import tpu_sc as plsc

VOCAB = 32000
RES = 2048
R_OUT = 512
BATCH = 16
T = 32
ROWS = BATCH * T

NC = 2
NS = 16
NW = NC * NS
B_PER_W = ROWS // NW

V_TILE = 3200


@functools.cache
def _make_gather():
    mesh = plsc.VectorSubcoreMesh(core_axis_name="c", subcore_axis_name="s")

    @functools.partial(
        pl.kernel,
        mesh=mesh,
        out_type=jax.ShapeDtypeStruct((ROWS, RES), jnp.float32),
        scratch_types=[
            pltpu.VMEM((B_PER_W,), jnp.int32),
            pltpu.VMEM((B_PER_W, RES), jnp.float32),
            pltpu.SemaphoreType.DMA,
        ],
    )
    def gather(table_hbm, idx_hbm, out_hbm, idx_v, rows_v, sem):
        wid = lax.axis_index("s") * NC + lax.axis_index("c")
        base = wid * B_PER_W
        pltpu.sync_copy(idx_hbm.at[pl.ds(base, B_PER_W)], idx_v)
        pltpu.async_copy(table_hbm.at[idx_v], rows_v, sem).wait()
        pltpu.sync_copy(rows_v, out_hbm.at[pl.ds(base, B_PER_W)])

    return gather


def _recurrence_body(u_ref, w_ref, a_ref, h0_ref, bw_ref, z_ref):
    a = a_ref[:]
    W = w_ref[:]
    Bw = bw_ref[:]
    h = jnp.broadcast_to(h0_ref[:], (BATCH, RES))

    def step(t, h):
        u = u_ref[pl.ds(t * BATCH, BATCH), :]
        rec = lax.dot_general(h, W, (((1,), (1,)), ((), ())),
                              preferred_element_type=jnp.float32)
        pre = jnp.clip(u + rec, -10.0, 10.0)
        hn = (1.0 - a) * h + a * jnp.tanh(pre)
        z = lax.dot_general(hn, Bw, (((1,), (1,)), ((), ())),
                            preferred_element_type=jnp.float32)
        z_ref[pl.ds(t, 1), :, :] = z[None, :, :]
        return hn

    lax.fori_loop(0, T, step, h)


def _readout_body(z_ref, aw_ref, ab_ref, o_ref):
    o_ref[:] = lax.dot_general(z_ref[:], aw_ref[:], (((1,), (1,)), ((), ())),
                               preferred_element_type=jnp.float32) + ab_ref[:]


def kernel(x, W_in, W_rec, a, B_w, A_w, A_b, h0):
    idx = x.astype(jnp.int32).T.reshape(ROWS)
    U = _make_gather()(W_in, idx)

    Z = pl.pallas_call(
        _recurrence_body,
        out_shape=jax.ShapeDtypeStruct((T, BATCH, R_OUT), jnp.float32),
    )(U, W_rec, a.reshape(1, RES), h0.reshape(1, RES), B_w)

    Zf = jnp.transpose(Z, (1, 0, 2)).reshape(ROWS, R_OUT)

    logits = pl.pallas_call(
        _readout_body,
        grid=(VOCAB // V_TILE,),
        in_specs=[
            pl.BlockSpec((ROWS, R_OUT), lambda i: (0, 0)),
            pl.BlockSpec((V_TILE, R_OUT), lambda i: (i, 0)),
            pl.BlockSpec((1, V_TILE), lambda i: (0, i)),
        ],
        out_specs=pl.BlockSpec((ROWS, V_TILE), lambda i: (0, i)),
        out_shape=jax.ShapeDtypeStruct((ROWS, VOCAB), jnp.float32),
    )(Zf, A_w, A_b.reshape(1, VOCAB))

    return logits.reshape(BATCH, T, VOCAB)


# speedup vs baseline: 15.1499x; 1.0156x over previous
import functools

import jax
import jax.numpy as jnp
from jax import lax
from jax.experimental import pallas as pl
from jax.experimental.pallas import tpu as pltpu
from jax.experimental.pallas import tpu_sc as plsc

VOCAB = 32000
RES = 2048
R_OUT = 512
BATCH = 16
T = 32
ROWS = BATCH * T

NC = 2
NS = 16
NW = NC * NS
B_PER_W = ROWS // NW

V_TILE = 3200


@functools.cache
def _make_gather():
    mesh = plsc.VectorSubcoreMesh(core_axis_name="c", subcore_axis_name="s")

    @functools.partial(
        pl.kernel,
        mesh=mesh,
        out_type=jax.ShapeDtypeStruct((ROWS, RES), jnp.float32),
        scratch_types=[
            pltpu.VMEM((B_PER_W,), jnp.int32),
            pltpu.VMEM((B_PER_W, RES), jnp.float32),
            pltpu.SemaphoreType.DMA,
        ],
    )
    def gather(table_hbm, idx_hbm, out_hbm, idx_v, rows_v, sem):
        wid = lax.axis_index("s") * NC + lax.axis_index("c")
        base = wid * B_PER_W
        pltpu.sync_copy(idx_hbm.at[pl.ds(base, B_PER_W)], idx_v)
        pltpu.async_copy(table_hbm.at[idx_v], rows_v, sem).wait()
        pltpu.sync_copy(rows_v, out_hbm.at[pl.ds(base, B_PER_W)])

    return gather


def _recurrence_body(u_ref, w_ref, a_ref, h0_ref, bw_ref, z_ref):
    a = a_ref[:]
    W = w_ref[:]
    Bw = bw_ref[:]
    h = jnp.broadcast_to(h0_ref[:], (BATCH, RES))

    def step(t, h):
        u = u_ref[pl.ds(t * BATCH, BATCH), :]
        rec = lax.dot_general(h, W, (((1,), (1,)), ((), ())),
                              preferred_element_type=jnp.float32)
        pre = jnp.clip(u + rec, -10.0, 10.0)
        hn = (1.0 - a) * h + a * jnp.tanh(pre)
        z = lax.dot_general(hn, Bw, (((1,), (1,)), ((), ())),
                            preferred_element_type=jnp.float32)
        z_ref[:, pl.ds(t, 1), :] = z[:, None, :]
        return hn

    lax.fori_loop(0, T, step, h)


def _readout_body(z_ref, aw_ref, ab_ref, o_ref):
    o_ref[:] = lax.dot_general(z_ref[:], aw_ref[:], (((1,), (1,)), ((), ())),
                               preferred_element_type=jnp.float32) + ab_ref[:]


def kernel(x, W_in, W_rec, a, B_w, A_w, A_b, h0):
    idx = x.astype(jnp.int32).T.reshape(ROWS)
    U = _make_gather()(W_in, idx)

    Z = pl.pallas_call(
        _recurrence_body,
        out_shape=jax.ShapeDtypeStruct((BATCH, T, R_OUT), jnp.float32),
    )(U, W_rec, a.reshape(1, RES), h0.reshape(1, RES), B_w)

    Zf = Z.reshape(ROWS, R_OUT)

    logits = pl.pallas_call(
        _readout_body,
        grid=(VOCAB // V_TILE,),
        in_specs=[
            pl.BlockSpec((ROWS, R_OUT), lambda i: (0, 0)),
            pl.BlockSpec((V_TILE, R_OUT), lambda i: (i, 0)),
            pl.BlockSpec((1, V_TILE), lambda i: (0, i)),
        ],
        out_specs=pl.BlockSpec((ROWS, V_TILE), lambda i: (0, i)),
        out_shape=jax.ShapeDtypeStruct((ROWS, VOCAB), jnp.float32),
    )(Zf, A_w, A_b.reshape(1, VOCAB))

    return logits.reshape(BATCH, T, VOCAB)


# speedup vs baseline: 15.3200x; 1.0112x over previous
import functools

import jax
import jax.numpy as jnp
from jax import lax
from jax.experimental import pallas as pl
from jax.experimental.pallas import tpu as pltpu
from jax.experimental.pallas import tpu_sc as plsc

VOCAB = 32000
RES = 2048
R_OUT = 512
BATCH = 16
T = 32
ROWS = BATCH * T

NC = 2
NS = 16
NW = NC * NS
B_PER_W = ROWS // NW

V_TILE = 3200
N_VT = VOCAB // V_TILE
HALF = T // 2
NBUF = 2


def _make_gather():
    mesh = plsc.VectorSubcoreMesh(core_axis_name="c", subcore_axis_name="s")

    @functools.partial(
        pl.kernel,
        mesh=mesh,
        out_type=jax.ShapeDtypeStruct((ROWS, RES), jnp.float32),
        scratch_types=[
            pltpu.VMEM((B_PER_W,), jnp.int32),
            pltpu.VMEM((B_PER_W, RES), jnp.float32),
            pltpu.SemaphoreType.DMA,
        ],
    )
    def gather(table_hbm, idx_hbm, out_hbm, idx_v, rows_v, sem):
        wid = lax.axis_index("s") * NC + lax.axis_index("c")
        base = wid * B_PER_W
        pltpu.sync_copy(idx_hbm.at[pl.ds(base, B_PER_W)], idx_v)
        pltpu.async_copy(table_hbm.at[idx_v], rows_v, sem).wait()
        pltpu.sync_copy(rows_v, out_hbm.at[pl.ds(base, B_PER_W)])

    return gather


_make_gather = functools.cache(_make_gather)


def _fused_body(u_ref, w_ref, a_ref, h0_ref, bw_ref, ab_ref, aw_ref,
                out_ref, zsc, abuf, obuf, asem, osem):
    def step(t, h):
        a = a_ref[:]
        u = u_ref[pl.ds(t * BATCH, BATCH), :]
        rec = lax.dot_general(h, w_ref[:], (((1,), (1,)), ((), ())),
                              preferred_element_type=jnp.float32)
        pre = jnp.clip(u + rec, -10.0, 10.0)
        hn = (1.0 - a) * h + a * jnp.tanh(pre)
        z = lax.dot_general(hn, bw_ref[:], (((1,), (1,)), ((), ())),
                            preferred_element_type=jnp.float32)
        zsc[:, pl.ds(t, 1), :] = z[:, None, :]
        return hn

    def a_copy(k):
        return pltpu.make_async_copy(
            aw_ref.at[pl.ds((k % N_VT) * V_TILE, V_TILE), :],
            abuf.at[k % NBUF],
            asem.at[k % NBUF])

    def o_copy(k, half):
        return pltpu.make_async_copy(
            obuf.at[k % NBUF],
            out_ref.at[:, pl.ds(half * HALF, HALF),
                       pl.ds((k % N_VT) * V_TILE, V_TILE)],
            osem.at[k % NBUF])

    def ro_tile(k, half):
        zh = zsc[:, pl.ds(half * HALF, HALF), :]
        A = abuf[k % NBUF]
        o = lax.dot_general(zh, A, (((2,), (1,)), ((), ())),
                            preferred_element_type=jnp.float32)
        bias = ab_ref[:, pl.ds((k % N_VT) * V_TILE, V_TILE)]
        obuf[k % NBUF] = o + bias[None, :, :]

    a_copy(0).start()
    a_copy(1).start()

    h = jnp.broadcast_to(h0_ref[:], (BATCH, RES))
    h = lax.fori_loop(0, HALF, step, h)

    bounds = [HALF + (i * HALF) // N_VT for i in range(N_VT + 1)]
    for i in range(N_VT):
        if bounds[i + 1] > bounds[i]:
            h = lax.fori_loop(bounds[i], bounds[i + 1], step, h)
        if i >= NBUF:
            o_copy(i - NBUF, 0).wait()
        a_copy(i).wait()
        ro_tile(i, 0)
        o_copy(i, 0).start()
        if i + NBUF < 2 * N_VT:
            a_copy(i + NBUF).start()

    for i in range(N_VT, 2 * N_VT):
        o_copy(i - NBUF, 0 if i - NBUF < N_VT else 1).wait()
        a_copy(i).wait()
        ro_tile(i, 1)
        o_copy(i, 1).start()
        if i + NBUF < 2 * N_VT:
            a_copy(i + NBUF).start()

    o_copy(2 * N_VT - 2, 1).wait()
    o_copy(2 * N_VT - 1, 1).wait()


def kernel(x, W_in, W_rec, a, B_w, A_w, A_b, h0):
    idx = x.astype(jnp.int32).T.reshape(ROWS)
    U = _make_gather()(W_in, idx)

    logits = pl.pallas_call(
        _fused_body,
        in_specs=[
            pl.BlockSpec(memory_space=pltpu.MemorySpace.VMEM),
            pl.BlockSpec(memory_space=pltpu.MemorySpace.VMEM),
            pl.BlockSpec(memory_space=pltpu.MemorySpace.VMEM),
            pl.BlockSpec(memory_space=pltpu.MemorySpace.VMEM),
            pl.BlockSpec(memory_space=pltpu.MemorySpace.VMEM),
            pl.BlockSpec(memory_space=pltpu.MemorySpace.VMEM),
            pl.BlockSpec(memory_space=pltpu.MemorySpace.HBM),
        ],
        out_specs=pl.BlockSpec(memory_space=pltpu.MemorySpace.HBM),
        out_shape=jax.ShapeDtypeStruct((BATCH, T, VOCAB), jnp.float32),
        scratch_shapes=[
            pltpu.VMEM((BATCH, T, R_OUT), jnp.float32),
            pltpu.VMEM((NBUF, V_TILE, R_OUT), jnp.float32),
            pltpu.VMEM((NBUF, BATCH, HALF, V_TILE), jnp.float32),
            pltpu.SemaphoreType.DMA((NBUF,)),
            pltpu.SemaphoreType.DMA((NBUF,)),
        ],
    )(U, W_rec, a.reshape(1, RES), h0.reshape(1, RES), B_w,
      A_b.reshape(1, VOCAB), A_w)

    return logits


# speedup vs baseline: 16.7111x; 1.0908x over previous
import functools

import jax
import jax.numpy as jnp
from jax import lax
from jax.experimental import pallas as pl
from jax.experimental.pallas import tpu as pltpu
from jax.experimental.pallas import tpu_sc as plsc

VOCAB = 32000
RES = 2048
R_OUT = 512
BATCH = 16
T = 32
ROWS = BATCH * T

NC = 2
NS = 16
NW = NC * NS
B_PER_W = ROWS // NW

V_TILE = 3200
N_VT = VOCAB // V_TILE
S0 = 24
S1 = T - S0
NBUF = 2


def _make_gather():
    mesh = plsc.VectorSubcoreMesh(core_axis_name="c", subcore_axis_name="s")

    @functools.partial(
        pl.kernel,
        mesh=mesh,
        out_type=jax.ShapeDtypeStruct((ROWS, RES), jnp.float32),
        scratch_types=[
            pltpu.VMEM((B_PER_W,), jnp.int32),
            pltpu.VMEM((B_PER_W, RES), jnp.float32),
            pltpu.SemaphoreType.DMA,
        ],
    )
    def gather(table_hbm, idx_hbm, out_hbm, idx_v, rows_v, sem):
        wid = lax.axis_index("s") * NC + lax.axis_index("c")
        base = wid * B_PER_W
        pltpu.sync_copy(idx_hbm.at[pl.ds(base, B_PER_W)], idx_v)
        pltpu.async_copy(table_hbm.at[idx_v], rows_v, sem).wait()
        pltpu.sync_copy(rows_v, out_hbm.at[pl.ds(base, B_PER_W)])

    return gather


_make_gather = functools.cache(_make_gather)


def _fused_body(u_ref, w_ref, a_ref, h0_ref, bw_ref, ab_ref, aw_ref,
                out_ref, hsc, zsc, abuf, obuf, asem, osem):
    def step(t, h):
        a = a_ref[:]
        u = u_ref[pl.ds(t * BATCH, BATCH), :]
        rec = lax.dot_general(h, w_ref[:], (((1,), (1,)), ((), ())),
                              preferred_element_type=jnp.float32)
        pre = jnp.clip(u + rec, -10.0, 10.0)
        hn = (1.0 - a) * h + a * jnp.tanh(pre)
        hsc[:, pl.ds(t, 1), :] = hn[:, None, :]
        return hn

    def z_block(toff, tlen):
        hs = hsc[:, pl.ds(toff, tlen), :]
        zv = lax.dot_general(hs, bw_ref[:], (((2,), (1,)), ((), ())),
                             preferred_element_type=jnp.float32)
        zsc[:, pl.ds(toff, tlen), :] = zv

    def a_copy(k):
        return pltpu.make_async_copy(
            aw_ref.at[pl.ds((k % N_VT) * V_TILE, V_TILE), :],
            abuf.at[k % NBUF],
            asem.at[k % NBUF])

    def o_copy(k, toff, tlen):
        return pltpu.make_async_copy(
            obuf.at[k % NBUF, :, pl.ds(0, tlen), :],
            out_ref.at[:, pl.ds(toff, tlen),
                       pl.ds((k % N_VT) * V_TILE, V_TILE)],
            osem.at[k % NBUF])

    def ro_tile(k, toff, tlen):
        zh = zsc[:, pl.ds(toff, tlen), :]
        A = abuf[k % NBUF]
        o = lax.dot_general(zh, A, (((2,), (1,)), ((), ())),
                            preferred_element_type=jnp.float32)
        bias = ab_ref[:, pl.ds((k % N_VT) * V_TILE, V_TILE)]
        obuf[k % NBUF, :, pl.ds(0, tlen), :] = o + bias[None, :, :]

    a_copy(0).start()
    a_copy(1).start()

    h = jnp.broadcast_to(h0_ref[:], (BATCH, RES))
    h = lax.fori_loop(0, S0, step, h)
    z_block(0, S0)

    bounds = [S0 + (i * S1) // N_VT for i in range(N_VT + 1)]
    for i in range(N_VT):
        if bounds[i + 1] > bounds[i]:
            h = lax.fori_loop(bounds[i], bounds[i + 1], step, h)
        if i >= NBUF:
            o_copy(i - NBUF, 0, S0).wait()
        a_copy(i).wait()
        ro_tile(i, 0, S0)
        o_copy(i, 0, S0).start()
        if i + NBUF < 2 * N_VT:
            a_copy(i + NBUF).start()

    z_block(S0, S1)
    for i in range(N_VT, 2 * N_VT):
        if i - NBUF < N_VT:
            o_copy(i - NBUF, 0, S0).wait()
        else:
            o_copy(i - NBUF, S0, S1).wait()
        a_copy(i).wait()
        ro_tile(i, S0, S1)
        o_copy(i, S0, S1).start()
        if i + NBUF < 2 * N_VT:
            a_copy(i + NBUF).start()

    o_copy(2 * N_VT - 2, S0, S1).wait()
    o_copy(2 * N_VT - 1, S0, S1).wait()


def kernel(x, W_in, W_rec, a, B_w, A_w, A_b, h0):
    idx = x.astype(jnp.int32).T.reshape(ROWS)
    U = _make_gather()(W_in, idx)

    logits = pl.pallas_call(
        _fused_body,
        in_specs=[
            pl.BlockSpec(memory_space=pltpu.MemorySpace.VMEM),
            pl.BlockSpec(memory_space=pltpu.MemorySpace.VMEM),
            pl.BlockSpec(memory_space=pltpu.MemorySpace.VMEM),
            pl.BlockSpec(memory_space=pltpu.MemorySpace.VMEM),
            pl.BlockSpec(memory_space=pltpu.MemorySpace.VMEM),
            pl.BlockSpec(memory_space=pltpu.MemorySpace.VMEM),
            pl.BlockSpec(memory_space=pltpu.MemorySpace.HBM),
        ],
        out_specs=pl.BlockSpec(memory_space=pltpu.MemorySpace.HBM),
        out_shape=jax.ShapeDtypeStruct((BATCH, T, VOCAB), jnp.float32),
        scratch_shapes=[
            pltpu.VMEM((BATCH, T, RES), jnp.float32),
            pltpu.VMEM((BATCH, T, R_OUT), jnp.float32),
            pltpu.VMEM((NBUF, V_TILE, R_OUT), jnp.float32),
            pltpu.VMEM((NBUF, BATCH, S0, V_TILE), jnp.float32),
            pltpu.SemaphoreType.DMA((NBUF,)),
            pltpu.SemaphoreType.DMA((NBUF,)),
        ],
    )(U, W_rec, a.reshape(1, RES), h0.reshape(1, RES), B_w,
      A_b.reshape(1, VOCAB), A_w)

    return logits


# speedup vs baseline: 16.8262x; 1.0069x over previous
import functools

import jax
import jax.numpy as jnp
from jax import lax
from jax.experimental import pallas as pl
from jax.experimental.pallas import tpu as pltpu
from jax.experimental.pallas import tpu_sc as plsc

VOCAB = 32000
RES = 2048
R_OUT = 512
BATCH = 16
T = 32
ROWS = BATCH * T

NC = 2
NS = 16
NW = NC * NS
B_PER_W = ROWS // NW

V_TILE = 3200
N_VT = VOCAB // V_TILE
S0 = 24
S1 = T - S0
NBUF = 2


def _make_gather():
    mesh = plsc.VectorSubcoreMesh(core_axis_name="c", subcore_axis_name="s")

    @functools.partial(
        pl.kernel,
        mesh=mesh,
        out_type=jax.ShapeDtypeStruct((ROWS, RES), jnp.float32),
        scratch_types=[
            pltpu.VMEM((B_PER_W,), jnp.int32),
            pltpu.VMEM((B_PER_W, RES), jnp.float32),
            pltpu.SemaphoreType.DMA,
        ],
    )
    def gather(table_hbm, idx_hbm, out_hbm, idx_v, rows_v, sem):
        wid = lax.axis_index("s") * NC + lax.axis_index("c")
        base = wid * B_PER_W
        pltpu.sync_copy(idx_hbm.at[pl.ds(base, B_PER_W)], idx_v)
        pltpu.async_copy(table_hbm.at[idx_v], rows_v, sem).wait()
        pltpu.sync_copy(rows_v, out_hbm.at[pl.ds(base, B_PER_W)])

    return gather


_make_gather = functools.cache(_make_gather)


def _fused_body(u_ref, w_ref, a_ref, h0_ref, bw_ref, ab_ref, aw_ref,
                out_ref, hsc, zsc, abuf, obuf, asem, osem):
    def step(t, h):
        a = a_ref[:]
        u = u_ref[pl.ds(t * BATCH, BATCH), :]
        rec = lax.dot_general(h, w_ref[:], (((1,), (1,)), ((), ())),
                              preferred_element_type=jnp.float32)
        pre = jnp.clip(u + rec, -10.0, 10.0)
        hn = (1.0 - a) * h + a * jnp.tanh(pre)
        hsc[:, pl.ds(t, 1), :] = hn[:, None, :]
        return hn

    def z_block(toff, tlen):
        hs = hsc[:, pl.ds(toff, tlen), :]
        zv = lax.dot_general(hs, bw_ref[:], (((2,), (1,)), ((), ())),
                             preferred_element_type=jnp.float32)
        zsc[:, pl.ds(toff, tlen), :] = zv

    def a_copy(k):
        return pltpu.make_async_copy(
            aw_ref.at[pl.ds((k % N_VT) * V_TILE, V_TILE), :],
            abuf.at[k % NBUF],
            asem.at[k % NBUF])

    def o_copy(k, toff, tlen):
        return pltpu.make_async_copy(
            obuf.at[k % NBUF, :, pl.ds(0, tlen), :],
            out_ref.at[:, pl.ds(toff, tlen),
                       pl.ds((k % N_VT) * V_TILE, V_TILE)],
            osem.at[k % NBUF])

    def ro_tile(k, toff, tlen):
        zh = zsc[:, pl.ds(toff, tlen), :]
        A = abuf[k % NBUF]
        o = lax.dot_general(zh, A, (((2,), (1,)), ((), ())),
                            preferred_element_type=jnp.float32)
        bias = ab_ref[:, pl.ds((k % N_VT) * V_TILE, V_TILE)]
        obuf[k % NBUF, :, pl.ds(0, tlen), :] = o + bias[None, :, :]

    a_copy(0).start()
    a_copy(1).start()

    h = jnp.broadcast_to(h0_ref[:], (BATCH, RES))
    h = lax.fori_loop(0, S0 // 2, lambda i, hh: step(2 * i + 1, step(2 * i, hh)), h)
    z_block(0, S0)

    bounds = [S0 + (i * S1) // N_VT for i in range(N_VT + 1)]
    for i in range(N_VT):
        if bounds[i + 1] > bounds[i]:
            h = lax.fori_loop(bounds[i], bounds[i + 1], step, h)
        if i >= NBUF:
            o_copy(i - NBUF, 0, S0).wait()
        a_copy(i).wait()
        ro_tile(i, 0, S0)
        o_copy(i, 0, S0).start()
        if i + NBUF < 2 * N_VT:
            a_copy(i + NBUF).start()

    z_block(S0, S1)
    for i in range(N_VT, 2 * N_VT):
        if i - NBUF < N_VT:
            o_copy(i - NBUF, 0, S0).wait()
        else:
            o_copy(i - NBUF, S0, S1).wait()
        a_copy(i).wait()
        ro_tile(i, S0, S1)
        o_copy(i, S0, S1).start()
        if i + NBUF < 2 * N_VT:
            a_copy(i + NBUF).start()

    o_copy(2 * N_VT - 2, S0, S1).wait()
    o_copy(2 * N_VT - 1, S0, S1).wait()


def kernel(x, W_in, W_rec, a, B_w, A_w, A_b, h0):
    idx = x.astype(jnp.int32).T.reshape(ROWS)
    U = _make_gather()(W_in, idx)

    logits = pl.pallas_call(
        _fused_body,
        in_specs=[
            pl.BlockSpec(memory_space=pltpu.MemorySpace.VMEM),
            pl.BlockSpec(memory_space=pltpu.MemorySpace.VMEM),
            pl.BlockSpec(memory_space=pltpu.MemorySpace.VMEM),
            pl.BlockSpec(memory_space=pltpu.MemorySpace.VMEM),
            pl.BlockSpec(memory_space=pltpu.MemorySpace.VMEM),
            pl.BlockSpec(memory_space=pltpu.MemorySpace.VMEM),
            pl.BlockSpec(memory_space=pltpu.MemorySpace.HBM),
        ],
        out_specs=pl.BlockSpec(memory_space=pltpu.MemorySpace.HBM),
        out_shape=jax.ShapeDtypeStruct((BATCH, T, VOCAB), jnp.float32),
        scratch_shapes=[
            pltpu.VMEM((BATCH, T, RES), jnp.float32),
            pltpu.VMEM((BATCH, T, R_OUT), jnp.float32),
            pltpu.VMEM((NBUF, V_TILE, R_OUT), jnp.float32),
            pltpu.VMEM((NBUF, BATCH, S0, V_TILE), jnp.float32),
            pltpu.SemaphoreType.DMA((NBUF,)),
            pltpu.SemaphoreType.DMA((NBUF,)),
        ],
    )(U, W_rec, a.reshape(1, RES), h0.reshape(1, RES), B_w,
      A_b.reshape(1, VOCAB), A_w)

    return logits


# speedup vs baseline: 17.2965x; 1.0279x over previous
import functools

import jax
import jax.numpy as jnp
from jax import lax
from jax.experimental import pallas as pl
from jax.experimental.pallas import tpu as pltpu
from jax.experimental.pallas import tpu_sc as plsc

VOCAB = 32000
RES = 2048
R_OUT = 512
BATCH = 16
T = 32
ROWS = BATCH * T

NC = 2
NS = 16
NW = NC * NS
B_PER_W = ROWS // NW

V_TILE = 3200
N_VT = VOCAB // V_TILE
S0 = 24
S1 = T - S0
ABUF_N = 3
OBUF_N = 2


def _make_gather():
    mesh = plsc.VectorSubcoreMesh(core_axis_name="c", subcore_axis_name="s")

    @functools.partial(
        pl.kernel,
        mesh=mesh,
        out_type=jax.ShapeDtypeStruct((ROWS, RES), jnp.float32),
        scratch_types=[
            pltpu.VMEM((B_PER_W // 2,), jnp.int32),
            pltpu.VMEM((B_PER_W // 2,), jnp.int32),
            pltpu.VMEM((B_PER_W // 2, RES), jnp.float32),
            pltpu.VMEM((B_PER_W // 2, RES), jnp.float32),
            pltpu.SemaphoreType.DMA,
            pltpu.SemaphoreType.DMA,
        ],
    )
    def gather(table_hbm, idx_hbm, out_hbm, idx_a, idx_b, rows_a, rows_b,
               sem_a, sem_b):
        wid = lax.axis_index("s") * NC + lax.axis_index("c")
        half = B_PER_W // 2
        base = wid * B_PER_W
        pltpu.sync_copy(idx_hbm.at[pl.ds(base, half)], idx_a)
        pltpu.sync_copy(idx_hbm.at[pl.ds(base + half, half)], idx_b)
        ga = pltpu.async_copy(table_hbm.at[idx_a], rows_a, sem_a)
        gb = pltpu.async_copy(table_hbm.at[idx_b], rows_b, sem_b)
        ga.wait()
        pltpu.sync_copy(rows_a, out_hbm.at[pl.ds(base, half)])
        gb.wait()
        pltpu.sync_copy(rows_b, out_hbm.at[pl.ds(base + half, half)])

    return gather


_make_gather = functools.cache(_make_gather)


def _fused_body(u_ref, w_ref, a_ref, h0_ref, bw_ref, ab_ref, aw_ref,
                out_ref, hsc, zsc, abuf, obuf, asem, osem):
    def step(t, h):
        a = a_ref[:]
        u = u_ref[pl.ds(t * BATCH, BATCH), :]
        rec = lax.dot_general(h, w_ref[:], (((1,), (1,)), ((), ())),
                              preferred_element_type=jnp.float32)
        pre = jnp.clip(u + rec, -10.0, 10.0)
        hn = (1.0 - a) * h + a * jnp.tanh(pre)
        hsc[:, pl.ds(t, 1), :] = hn[:, None, :]
        return hn

    def z_block(toff, tlen):
        hs = hsc[:, pl.ds(toff, tlen), :]
        zv = lax.dot_general(hs, bw_ref[:], (((2,), (1,)), ((), ())),
                             preferred_element_type=jnp.float32)
        zsc[:, pl.ds(toff, tlen), :] = zv

    def a_copy(k):
        return pltpu.make_async_copy(
            aw_ref.at[pl.ds((k % N_VT) * V_TILE, V_TILE), :],
            abuf.at[k % ABUF_N],
            asem.at[k % ABUF_N])

    def o_copy(k, toff, tlen):
        return pltpu.make_async_copy(
            obuf.at[k % OBUF_N, :, pl.ds(0, tlen), :],
            out_ref.at[:, pl.ds(toff, tlen),
                       pl.ds((k % N_VT) * V_TILE, V_TILE)],
            osem.at[k % OBUF_N])

    def ro_tile(k, toff, tlen):
        zh = zsc[:, pl.ds(toff, tlen), :]
        A = abuf[k % ABUF_N]
        o = lax.dot_general(zh, A, (((2,), (1,)), ((), ())),
                            preferred_element_type=jnp.float32)
        bias = ab_ref[:, pl.ds((k % N_VT) * V_TILE, V_TILE)]
        obuf[k % OBUF_N, :, pl.ds(0, tlen), :] = o + bias[None, :, :]

    for k in range(ABUF_N):
        a_copy(k).start()

    h = jnp.broadcast_to(h0_ref[:], (BATCH, RES))
    h = lax.fori_loop(0, S0 // 2, lambda i, hh: step(2 * i + 1, step(2 * i, hh)), h)
    z_block(0, S0)

    bounds = [S0 + (i * S1) // N_VT for i in range(N_VT + 1)]
    for i in range(N_VT):
        if bounds[i + 1] > bounds[i]:
            h = lax.fori_loop(bounds[i], bounds[i + 1], step, h)
        if i >= OBUF_N:
            o_copy(i - OBUF_N, 0, S0).wait()
        a_copy(i).wait()
        ro_tile(i, 0, S0)
        o_copy(i, 0, S0).start()
        if i + ABUF_N < 2 * N_VT:
            a_copy(i + ABUF_N).start()

    z_block(S0, S1)
    for i in range(N_VT, 2 * N_VT):
        if i - OBUF_N < N_VT:
            o_copy(i - OBUF_N, 0, S0).wait()
        else:
            o_copy(i - OBUF_N, S0, S1).wait()
        a_copy(i).wait()
        ro_tile(i, S0, S1)
        o_copy(i, S0, S1).start()
        if i + ABUF_N < 2 * N_VT:
            a_copy(i + ABUF_N).start()

    o_copy(2 * N_VT - 2, S0, S1).wait()
    o_copy(2 * N_VT - 1, S0, S1).wait()


def kernel(x, W_in, W_rec, a, B_w, A_w, A_b, h0):
    idx = x.astype(jnp.int32).T.reshape(ROWS)
    U = _make_gather()(W_in, idx)

    logits = pl.pallas_call(
        _fused_body,
        in_specs=[
            pl.BlockSpec(memory_space=pltpu.MemorySpace.VMEM),
            pl.BlockSpec(memory_space=pltpu.MemorySpace.VMEM),
            pl.BlockSpec(memory_space=pltpu.MemorySpace.VMEM),
            pl.BlockSpec(memory_space=pltpu.MemorySpace.VMEM),
            pl.BlockSpec(memory_space=pltpu.MemorySpace.VMEM),
            pl.BlockSpec(memory_space=pltpu.MemorySpace.VMEM),
            pl.BlockSpec(memory_space=pltpu.MemorySpace.HBM),
        ],
        out_specs=pl.BlockSpec(memory_space=pltpu.MemorySpace.HBM),
        out_shape=jax.ShapeDtypeStruct((BATCH, T, VOCAB), jnp.float32),
        scratch_shapes=[
            pltpu.VMEM((BATCH, T, RES), jnp.float32),
            pltpu.VMEM((BATCH, T, R_OUT), jnp.float32),
            pltpu.VMEM((ABUF_N, V_TILE, R_OUT), jnp.float32),
            pltpu.VMEM((OBUF_N, BATCH, S0, V_TILE), jnp.float32),
            pltpu.SemaphoreType.DMA((ABUF_N,)),
            pltpu.SemaphoreType.DMA((OBUF_N,)),
        ],
    )(U, W_rec, a.reshape(1, RES), h0.reshape(1, RES), B_w,
      A_b.reshape(1, VOCAB), A_w)

    return logits
